# baseline (speedup 1.0000x reference)
import numpy as np
import jax
import jax.numpy as jnp
from jax.experimental import pallas as pl
from jax.experimental.pallas import tpu as pltpu


def _vfull(shape):
    n = len(shape)
    return pl.BlockSpec(tuple(shape), lambda i, _n=n: (0,) * _n)


def kernel(x, w_conv1, b_conv1, w_conv2, b_conv2,
           w_fc1, b_fc1, w_fc2, b_fc2, w_fc3, b_fc3):
    f32 = jnp.float32
    x = x.astype(f32)
    B, cin, H, W = x.shape
    cmid = w_conv1.shape[0]

    pad = 2
    Hp, Wp = H + 2 * pad, W + 2 * pad
    H1, W1 = Hp - 1, Wp - 1
    H1p, W1p = H1 // 2, W1 // 2
    H2, W2 = H1p - 1, W1p - 1
    H2p, W2p = H2 // 2, W2 // 2
    feat = cmid * H2p * W2p
    n_h1, n_h2, n_out = w_fc1.shape[1], w_fc2.shape[1], w_fc3.shape[1]

    CB1 = B * Hp
    M1 = cmid * CB1
    SC2 = 16
    M2 = cmid * SC2
    K2 = M1 - 1
    SLAB = 8
    assert B * H1p <= SC2 and B <= SLAB

    Q = np.zeros((2, CB1, CB1), np.float32)
    vrow1 = np.zeros((CB1,), np.float32)
    for b in range(B):
        for h in range(H1):
            for dy in range(2):
                Q[dy, b * Hp + h, b * Hp + h + dy] = 1.0
            vrow1[b * Hp + h] = 1.0
    P = np.zeros((2, SC2, CB1), np.float32)
    v2row = np.zeros((SC2,), np.float32)
    for b in range(B):
        for h2 in range(H2):
            for dy in range(2):
                P[dy, b * H1p + h2, b * Hp + 2 * (h2 + dy)] = 1.0
            v2row[b * H1p + h2] = 1.0
    s1c = np.zeros((W1 - 1, W1p), np.float32)
    s1c[2 * np.arange(W1p), np.arange(W1p)] = 1.0
    s2c = np.zeros((W2 - 1, W2p), np.float32)
    s2c[2 * np.arange(W2p), np.arange(W2p)] = 1.0
    s2r = np.zeros((cmid * H2p * SLAB, M2 - 1), np.float32)
    for c in range(cmid):
        for y in range(H2p):
            for b in range(B):
                s2r[(c * H2p + y) * SLAB + b, c * SC2 + b * H1p + 2 * y] = 1.0

    w1 = w_conv1.astype(f32)
    w2 = w_conv2.astype(f32)
    a1_parts = []
    for dx in range(2):
        a = sum((w1[:, :, dy, dx][:, None, :, None] * Q[dy][None, :, None, :])
                for dy in range(2)).reshape(M1, cin * CB1)
        a1_parts.append(a)
    b1col = (b_conv1.astype(f32)[:, None] * vrow1[None, :]).reshape(M1, 1)
    a1 = jnp.concatenate(a1_parts + [b1col], axis=1)

    a2 = []
    for dx in range(2):
        a = sum((w2[:, :, dy, dx][:, None, :, None] * P[dy][None, :, None, :])
                for dy in range(2)).reshape(M2, cmid * CB1)[:, :K2]
        a2.append(a)
    b2col = (b_conv2.astype(f32)[:, None] * v2row[None, :]).reshape(M2, 1)

    xpad = jnp.pad(x, ((0, 0), (0, 0), (pad, pad), (pad, pad)))
    xs = jnp.transpose(xpad, (1, 0, 2, 3)).reshape(cin * B * Hp, Wp)
    xs2 = jnp.concatenate(
        [xs[:, 0:W1], xs[:, 1:W1 + 1], jnp.ones((1, W1), f32)], axis=0)

    n_slab = cmid * H2p

    def body(xs2_ref, a1_ref, a2a_ref, a2b_ref, b2_ref,
             s1c_ref, s2c_ref, s2r_ref,
             wf1_ref, bf1_ref, wf2_ref, bf2_ref, wf3_ref, bf3_ref,
             o_ref, fcin_ref):
        c1 = jnp.maximum(
            jnp.dot(a1_ref[...], xs2_ref[...], preferred_element_type=f32), 0.0)
        mw1 = jnp.maximum(c1[:, 0:W1 - 1], c1[:, 1:W1])
        p1c = jnp.dot(mw1, s1c_ref[...], preferred_element_type=f32)
        mh1 = jnp.maximum(p1c[:-1, :], p1c[1:, :])
        c2 = jnp.dot(a2a_ref[...], mh1[:, 0:W2], preferred_element_type=f32)
        c2 = c2 + jnp.dot(a2b_ref[...], mh1[:, 1:W1p],
                          preferred_element_type=f32)
        c2 = jnp.maximum(c2 + b2_ref[...], 0.0)
        mw2 = jnp.maximum(c2[:, 0:W2 - 1], c2[:, 1:W2])
        p2c = jnp.dot(mw2, s2c_ref[...], preferred_element_type=f32)
        mh2 = jnp.maximum(p2c[:-1, :], p2c[1:, :])
        g = jnp.dot(s2r_ref[...], mh2, preferred_element_type=f32)
        for i in range(n_slab):
            fcin_ref[:, i * W2p:(i + 1) * W2p] = g[i * SLAB:(i + 1) * SLAB, :]
        h = fcin_ref[...]
        h = jnp.maximum(
            jnp.dot(h, wf1_ref[...], preferred_element_type=f32)
            + bf1_ref[...], 0.0)
        h = jnp.maximum(
            jnp.dot(h, wf2_ref[...], preferred_element_type=f32)
            + bf2_ref[...], 0.0)
        o = jnp.dot(h, wf3_ref[...], preferred_element_type=f32) + bf3_ref[...]
        o_ref[...] = o[0:B, :].astype(o_ref.dtype)

    args = (
        xs2, a1, a2[0], a2[1], b2col,
        jnp.asarray(s1c), jnp.asarray(s2c), jnp.asarray(s2r),
        w_fc1.astype(f32), b_fc1.astype(f32).reshape(1, -1),
        w_fc2.astype(f32), b_fc2.astype(f32).reshape(1, -1),
        w_fc3.astype(f32), b_fc3.astype(f32).reshape(1, -1),
    )
    return pl.pallas_call(
        body,
        out_shape=jax.ShapeDtypeStruct((B, n_out), f32),
        grid=(1,),
        in_specs=[_vfull(a.shape) for a in args],
        out_specs=_vfull((B, n_out)),
        scratch_shapes=[pltpu.VMEM((SLAB, feat), f32)],
        compiler_params=pltpu.CompilerParams(dimension_semantics=("arbitrary",)),
    )(*args)


# speedup vs baseline: 1.6700x; 1.6700x over previous
import numpy as np
import jax
import jax.numpy as jnp
from jax.experimental import pallas as pl
from jax.experimental.pallas import tpu as pltpu


def _cfull(shape):
    n = len(shape)
    return pl.BlockSpec(tuple(shape), lambda i, _n=n: (0,) * _n)


def kernel(x, w_conv1, b_conv1, w_conv2, b_conv2,
           w_fc1, b_fc1, w_fc2, b_fc2, w_fc3, b_fc3):
    f32 = jnp.float32
    x = x.astype(f32)
    B, cin, H, W = x.shape
    cmid = w_conv1.shape[0]

    pad = 2
    Hp, Wp = H + 2 * pad, W + 2 * pad
    H1, W1 = Hp - 1, Wp - 1
    H1p, W1p = H1 // 2, W1 // 2
    H2, W2 = H1p - 1, W1p - 1
    H2p, W2p = H2 // 2, W2 // 2
    feat = cmid * H2p * W2p
    n_h1, n_h2, n_out = w_fc1.shape[1], w_fc2.shape[1], w_fc3.shape[1]

    CB1 = B * Hp
    M1 = cmid * CB1
    SC2 = 16
    M2 = cmid * SC2
    K2 = M1 - 1
    SLAB = 8
    n_slab = cmid * H2p
    assert B * H1p <= SC2 and B <= SLAB

    NB = 128
    G = n_h1 // NB
    assert n_h1 % NB == 0

    Q = np.zeros((2, CB1, CB1), np.float32)
    vrow1 = np.zeros((CB1,), np.float32)
    for b in range(B):
        for h in range(H1):
            for dy in range(2):
                Q[dy, b * Hp + h, b * Hp + h + dy] = 1.0
            vrow1[b * Hp + h] = 1.0
    P = np.zeros((2, SC2, CB1), np.float32)
    v2row = np.zeros((SC2,), np.float32)
    for b in range(B):
        for h2 in range(H2):
            for dy in range(2):
                P[dy, b * H1p + h2, b * Hp + 2 * (h2 + dy)] = 1.0
            v2row[b * H1p + h2] = 1.0
    TQ = np.concatenate([np.tile(Q[dy], (cmid, cin)) for dy in range(2)], 0)
    TP = np.concatenate(
        [np.tile(P[dy], (cmid, cmid))[:, :K2] for dy in range(2)], 0)
    vc1 = np.tile(vrow1[:, None], (cmid, 1))
    vc2 = np.tile(v2row[:, None], (cmid, 1))
    E1 = np.kron(np.eye(cmid, dtype=np.float32), np.ones((CB1, 1), np.float32))
    E2 = np.kron(np.eye(cmid, dtype=np.float32), np.ones((SC2, 1), np.float32))
    NT1, NT2 = cin * 4, cmid * 4
    F1 = np.zeros((4, NT1, cin * CB1), np.float32)
    F2 = np.zeros((4, NT2, K2), np.float32)
    for k in range(4):
        for ci in range(cin):
            F1[k, ci * 4 + k, ci * CB1:(ci + 1) * CB1] = 1.0
        for ci in range(cmid):
            F2[k, ci * 4 + k, ci * CB1:min((ci + 1) * CB1, K2)] = 1.0
    F1 = F1.reshape(4 * NT1, cin * CB1)
    F2 = F2.reshape(4 * NT2, K2)
    s1c = np.zeros((W1 - 1, W1p), np.float32)
    s1c[2 * np.arange(W1p), np.arange(W1p)] = 1.0
    s2c = np.zeros((W2 - 1, W2p), np.float32)
    s2c[2 * np.arange(W2p), np.arange(W2p)] = 1.0
    s2r = np.zeros((n_slab * SLAB, M2 - 1), np.float32)
    for c in range(cmid):
        for y in range(H2p):
            for b in range(B):
                s2r[(c * H2p + y) * SLAB + b, c * SC2 + b * H1p + 2 * y] = 1.0

    w1r = w_conv1.astype(f32).reshape(cmid, NT1)
    w2r = w_conv2.astype(f32).reshape(cmid, NT2)
    b1r = b_conv1.astype(f32).reshape(cmid, 1)
    b2r = b_conv2.astype(f32).reshape(cmid, 1)

    def body(x_ref, w1_ref, b1_ref, w2_ref, b2_ref,
             e1_ref, f1_ref, tq_ref, vc1_ref,
             e2_ref, f2_ref, tp_ref, vc2_ref,
             s1c_ref, s2c_ref, s2r_ref,
             wf1_ref, bf1_ref, wf2_ref, bf2_ref, wf3_ref, bf3_ref,
             o_ref, xs_sc, fcin_sc, h1_sc):
        i = pl.program_id(0)

        @pl.when(i == 0)
        def _stage1():
            xs_sc[...] = jnp.zeros((cin * CB1, Wp), f32)
            for ci in range(cin):
                for b in range(B):
                    xs_sc[ci * CB1 + b * Hp + pad:
                          ci * CB1 + b * Hp + pad + H,
                          pad:pad + W] = x_ref[b, ci]
            xs = xs_sc[...]
            wb1 = jnp.dot(e1_ref[...], w1_ref[...], preferred_element_type=f32)
            a1 = []
            for dx in range(2):
                a = sum(jnp.dot(wb1, f1_ref[(dy * 2 + dx) * NT1:
                                            (dy * 2 + dx + 1) * NT1, :],
                                preferred_element_type=f32)
                        * tq_ref[dy * M1:(dy + 1) * M1, :] for dy in range(2))
                a1.append(a)
            b1c = jnp.dot(e1_ref[...], b1_ref[...],
                          preferred_element_type=f32) * vc1_ref[...]
            c1 = jnp.maximum(
                jnp.dot(a1[0], xs[:, 0:W1], preferred_element_type=f32)
                + jnp.dot(a1[1], xs[:, 1:Wp], preferred_element_type=f32)
                + b1c, 0.0)
            mw1 = jnp.maximum(c1[:, 0:W1 - 1], c1[:, 1:W1])
            p1c = jnp.dot(mw1, s1c_ref[...], preferred_element_type=f32)
            mh1 = jnp.maximum(p1c[:-1, :], p1c[1:, :])
            wb2 = jnp.dot(e2_ref[...], w2_ref[...], preferred_element_type=f32)
            a2 = []
            for dx in range(2):
                a = sum(jnp.dot(wb2, f2_ref[(dy * 2 + dx) * NT2:
                                            (dy * 2 + dx + 1) * NT2, :],
                                preferred_element_type=f32)
                        * tp_ref[dy * M2:(dy + 1) * M2, :] for dy in range(2))
                a2.append(a)
            b2c = jnp.dot(e2_ref[...], b2_ref[...],
                          preferred_element_type=f32) * vc2_ref[...]
            c2 = jnp.maximum(
                jnp.dot(a2[0], mh1[:, 0:W2], preferred_element_type=f32)
                + jnp.dot(a2[1], mh1[:, 1:W1p], preferred_element_type=f32)
                + b2c, 0.0)
            mw2 = jnp.maximum(c2[:, 0:W2 - 1], c2[:, 1:W2])
            p2c = jnp.dot(mw2, s2c_ref[...], preferred_element_type=f32)
            mh2 = jnp.maximum(p2c[:-1, :], p2c[1:, :])
            g = jnp.dot(s2r_ref[...], mh2, preferred_element_type=f32)
            for t in range(n_slab):
                fcin_sc[:, t * W2p:(t + 1) * W2p] = \
                    g[t * SLAB:(t + 1) * SLAB, :]

        hb = jnp.maximum(
            jnp.dot(fcin_sc[...], wf1_ref[...], preferred_element_type=f32)
            + bf1_ref[...], 0.0)
        h1_sc[pl.ds(i * SLAB, SLAB), :] = hb

        @pl.when(i == G - 1)
        def _stage3():
            h2 = sum(jnp.dot(h1_sc[gg * SLAB:(gg + 1) * SLAB, :],
                             wf2_ref[gg * NB:(gg + 1) * NB, :],
                             preferred_element_type=f32) for gg in range(G))
            h2 = jnp.maximum(h2 + bf2_ref[...], 0.0)
            o = jnp.dot(h2, wf3_ref[...], preferred_element_type=f32) \
                + bf3_ref[...]
            o_ref[...] = o[0:B, :].astype(o_ref.dtype)

    args = (
        x, w1r, b1r, w2r, b2r,
        jnp.asarray(E1), jnp.asarray(F1), jnp.asarray(TQ), jnp.asarray(vc1),
        jnp.asarray(E2), jnp.asarray(F2), jnp.asarray(TP), jnp.asarray(vc2),
        jnp.asarray(s1c), jnp.asarray(s2c), jnp.asarray(s2r),
        w_fc1.astype(f32), b_fc1.astype(f32).reshape(1, -1),
        w_fc2.astype(f32), b_fc2.astype(f32).reshape(1, -1),
        w_fc3.astype(f32), b_fc3.astype(f32).reshape(1, -1),
    )
    in_specs = [_cfull(a.shape) for a in args]
    in_specs[16] = pl.BlockSpec((feat, NB), lambda i: (0, i))
    in_specs[17] = pl.BlockSpec((1, NB), lambda i: (0, i))
    return pl.pallas_call(
        body,
        out_shape=jax.ShapeDtypeStruct((B, n_out), f32),
        grid=(G,),
        in_specs=in_specs,
        out_specs=_cfull((B, n_out)),
        scratch_shapes=[
            pltpu.VMEM((cin * CB1, Wp), f32),
            pltpu.VMEM((SLAB, feat), f32),
            pltpu.VMEM((G * SLAB, NB), f32),
        ],
        compiler_params=pltpu.CompilerParams(
            dimension_semantics=("arbitrary",)),
    )(*args)


# speedup vs baseline: 1.7042x; 1.0205x over previous
import numpy as np
import jax
import jax.numpy as jnp
from jax.experimental import pallas as pl
from jax.experimental.pallas import tpu as pltpu


def _cfull(shape):
    n = len(shape)
    return pl.BlockSpec(tuple(shape), lambda i, _n=n: (0,) * _n)


def kernel(x, w_conv1, b_conv1, w_conv2, b_conv2,
           w_fc1, b_fc1, w_fc2, b_fc2, w_fc3, b_fc3):
    f32 = jnp.float32
    x = x.astype(f32)
    B, cin, H, W = x.shape
    cmid = w_conv1.shape[0]

    pad = 2
    Hp, Wp = H + 2 * pad, W + 2 * pad
    H1, W1 = Hp - 1, Wp - 1
    H1p, W1p = H1 // 2, W1 // 2
    H2, W2 = H1p - 1, W1p - 1
    H2p, W2p = H2 // 2, W2 // 2
    feat = cmid * H2p * W2p
    n_h1, n_h2, n_out = w_fc1.shape[1], w_fc2.shape[1], w_fc3.shape[1]

    CB1 = B * Hp
    M1 = cmid * CB1
    SC2 = 16
    M2 = cmid * SC2
    SLAB = 8
    n_slab = cmid * H2p
    assert B * H1p <= SC2 and B <= SLAB

    G = 8
    KB = feat // G
    assert feat % G == 0 and KB % 128 == 0

    Q = np.zeros((2, CB1, CB1), np.float32)
    vrow1 = np.zeros((CB1,), np.float32)
    for b in range(B):
        for h in range(H1):
            for dy in range(2):
                Q[dy, b * Hp + h, b * Hp + h + dy] = 1.0
            vrow1[b * Hp + h] = 1.0
    P = np.zeros((2, SC2, CB1), np.float32)
    v2row = np.zeros((SC2,), np.float32)
    for b in range(B):
        for h2 in range(H2):
            for dy in range(2):
                P[dy, b * H1p + h2, b * Hp + 2 * (h2 + dy)] = 1.0
            v2row[b * H1p + h2] = 1.0
    E1 = np.kron(np.eye(cmid, dtype=np.float32), np.ones((CB1, 1), np.float32))
    E2 = np.kron(np.eye(cmid, dtype=np.float32), np.ones((SC2, 1), np.float32))
    E1b = np.kron(np.ones((cmid, 1), np.float32), np.eye(CB1, dtype=np.float32))
    E2b = np.kron(np.ones((cmid, 1), np.float32), np.eye(SC2, dtype=np.float32))
    Ft1 = np.tile(np.eye(CB1, dtype=np.float32), (1, cin))
    Ft2 = np.tile(np.eye(CB1, dtype=np.float32), (1, cmid))
    Fs1 = np.kron(np.eye(cin, dtype=np.float32), np.ones((1, CB1), np.float32))
    Fs2 = np.kron(np.eye(cmid, dtype=np.float32), np.ones((1, CB1), np.float32))
    NT1, NT2 = cin * 4, cmid * 4
    S1 = np.zeros((4, NT1, cin), np.float32)
    S2 = np.zeros((4, NT2, cmid), np.float32)
    for k in range(4):
        for ci in range(cin):
            S1[k, ci * 4 + k, ci] = 1.0
        for ci in range(cmid):
            S2[k, ci * 4 + k, ci] = 1.0
    S1 = S1.reshape(4 * NT1, cin)
    S2 = S2.reshape(4 * NT2, cmid)
    QQ = Q.reshape(2 * CB1, CB1)
    PP = P.reshape(2 * SC2, CB1)
    vc1 = np.tile(vrow1[:, None], (cmid, 1))
    vc2 = np.tile(v2row[:, None], (cmid, 1))
    s1c = np.zeros((W1 - 1, W1p), np.float32)
    s1c[2 * np.arange(W1p), np.arange(W1p)] = 1.0
    s2c = np.zeros((W2 - 1, W2p), np.float32)
    s2c[2 * np.arange(W2p), np.arange(W2p)] = 1.0

    w1r = w_conv1.astype(f32).reshape(cmid, NT1)
    w2r = w_conv2.astype(f32).reshape(cmid, NT2)
    b1r = b_conv1.astype(f32).reshape(cmid, 1)
    b2r = b_conv2.astype(f32).reshape(cmid, 1)

    def body(x_ref, w1_ref, b1_ref, w2_ref, b2_ref,
             e1_ref, e2_ref, e1b_ref, e2b_ref,
             ft1_ref, ft2_ref, fs1_ref, fs2_ref,
             s1_ref, s2_ref, q_ref, p_ref, vc1_ref, vc2_ref,
             s1c_ref, s2c_ref,
             wf1_ref, bf1_ref, wf2_ref, bf2_ref, wf3_ref, bf3_ref,
             o_ref, xs_sc, fcin_sc, h1_sc):
        i = pl.program_id(0)

        def dot(a, b):
            return jnp.dot(a, b, preferred_element_type=f32)

        @pl.when(i == 0)
        def _stage1():
            xs_sc[...] = jnp.zeros((cin * CB1, Wp), f32)
            for ci in range(cin):
                for b in range(B):
                    xs_sc[ci * CB1 + b * Hp + pad:
                          ci * CB1 + b * Hp + pad + H,
                          pad:pad + W] = x_ref[b, ci]
            xs = xs_sc[...]
            wb1 = dot(e1_ref[...], w1_ref[...])
            msk1 = [dot(dot(e1b_ref[...], q_ref[dy * CB1:(dy + 1) * CB1, :]),
                        ft1_ref[...]) for dy in range(2)]
            a1 = []
            for dx in range(2):
                a = sum(dot(dot(wb1, s1_ref[(dy * 2 + dx) * NT1:
                                            (dy * 2 + dx + 1) * NT1, :]),
                            fs1_ref[...]) * msk1[dy] for dy in range(2))
                a1.append(a)
            b1c = dot(e1_ref[...], b1_ref[...]) * vc1_ref[...]
            c1 = jnp.maximum(dot(a1[0], xs[:, 0:W1])
                             + dot(a1[1], xs[:, 1:Wp]) + b1c, 0.0)
            mw1 = jnp.maximum(c1[:, 0:W1 - 1], c1[:, 1:W1])
            p1c = dot(mw1, s1c_ref[...])
            mh1 = jnp.maximum(
                p1c, jnp.concatenate([p1c[1:, :], p1c[M1 - 1:, :]], axis=0))
            wb2 = dot(e2_ref[...], w2_ref[...])
            msk2 = [dot(dot(e2b_ref[...], p_ref[dy * SC2:(dy + 1) * SC2, :]),
                        ft2_ref[...]) for dy in range(2)]
            a2 = []
            for dx in range(2):
                a = sum(dot(dot(wb2, s2_ref[(dy * 2 + dx) * NT2:
                                            (dy * 2 + dx + 1) * NT2, :]),
                            fs2_ref[...]) * msk2[dy] for dy in range(2))
                a2.append(a)
            b2c = dot(e2_ref[...], b2_ref[...]) * vc2_ref[...]
            c2 = jnp.maximum(dot(a2[0], mh1[:, 0:W2])
                             + dot(a2[1], mh1[:, 1:W1p]) + b2c, 0.0)
            mw2 = jnp.maximum(c2[:, 0:W2 - 1], c2[:, 1:W2])
            p2c = dot(mw2, s2c_ref[...])
            mh2 = jnp.maximum(p2c[:-1, :], p2c[1:, :])
            for c in range(cmid):
                for y in range(H2p):
                    t = c * H2p + y
                    for b in range(B):
                        r = c * SC2 + b * H1p + 2 * y
                        fcin_sc[b:b + 1, t * W2p:(t + 1) * W2p] = \
                            mh2[r:r + 1, :]
            h1_sc[...] = jnp.zeros((SLAB, n_h1), f32)

        h1_sc[...] += dot(fcin_sc[:, pl.ds(i * KB, KB)], wf1_ref[...])

        @pl.when(i == G - 1)
        def _stage3():
            h1 = jnp.maximum(h1_sc[...] + bf1_ref[...], 0.0)
            h2 = jnp.maximum(dot(h1, wf2_ref[...]) + bf2_ref[...], 0.0)
            o = dot(h2, wf3_ref[...]) + bf3_ref[...]
            o_ref[...] = o[0:B, :].astype(o_ref.dtype)

    args = (
        x, w1r, b1r, w2r, b2r,
        jnp.asarray(E1), jnp.asarray(E2), jnp.asarray(E1b), jnp.asarray(E2b),
        jnp.asarray(Ft1), jnp.asarray(Ft2), jnp.asarray(Fs1), jnp.asarray(Fs2),
        jnp.asarray(S1), jnp.asarray(S2), jnp.asarray(QQ), jnp.asarray(PP),
        jnp.asarray(vc1), jnp.asarray(vc2),
        jnp.asarray(s1c), jnp.asarray(s2c),
        w_fc1.astype(f32), b_fc1.astype(f32).reshape(1, -1),
        w_fc2.astype(f32), b_fc2.astype(f32).reshape(1, -1),
        w_fc3.astype(f32), b_fc3.astype(f32).reshape(1, -1),
    )
    in_specs = [_cfull(a.shape) for a in args]
    in_specs[21] = pl.BlockSpec((KB, n_h1), lambda i: (i, 0))
    return pl.pallas_call(
        body,
        out_shape=jax.ShapeDtypeStruct((B, n_out), f32),
        grid=(G,),
        in_specs=in_specs,
        out_specs=_cfull((B, n_out)),
        scratch_shapes=[
            pltpu.VMEM((cin * CB1, Wp), f32),
            pltpu.VMEM((SLAB, feat), f32),
            pltpu.VMEM((SLAB, n_h1), f32),
        ],
        compiler_params=pltpu.CompilerParams(
            dimension_semantics=("arbitrary",)),
    )(*args)
